# baseline (speedup 1.0000x reference)
import functools

import jax
import jax.numpy as jnp
from jax import lax
from jax.experimental import pallas as pl
from jax.experimental.pallas import tpu as pltpu
from jax.experimental.pallas import tpu_sc as plsc

_NPTS = 16384
_D = 32
_K = 8192
_M_BLOCK = 1024

_NC = 2
_NS = 16
_NW = _NC * _NS
_GATHER_CHUNK = 128


def _argmin_body(z_ref, cb_ref, out_ref, cnorm_ref):
    @pl.when(pl.program_id(0) == 0)
    def _():
        cb = cb_ref[...]
        cnorm_ref[...] = jnp.sum(cb * cb, axis=1, keepdims=True)

    zb2 = z_ref[0] * -2.0
    s = lax.dot_general(cb_ref[...], zb2, (((1,), (0,)), ((), ())),
                        preferred_element_type=jnp.float32)
    d = cnorm_ref[...] + s
    bi = jnp.argmin(d, axis=0)
    out_ref[...] = bi.astype(jnp.int32).reshape(1, 1, _M_BLOCK)


def _encode_indices(z3d, codebook, interpret=False):
    n_blocks = z3d.shape[0]
    out = pl.pallas_call(
        _argmin_body,
        grid=(n_blocks,),
        in_specs=[
            pl.BlockSpec((1, _D, _M_BLOCK), lambda g: (g, 0, 0)),
            pl.BlockSpec((_K, _D), lambda g: (0, 0)),
        ],
        out_specs=pl.BlockSpec((1, 1, _M_BLOCK), lambda g: (g, 0, 0)),
        out_shape=jax.ShapeDtypeStruct((n_blocks, 1, _M_BLOCK), jnp.int32),
        scratch_shapes=[pltpu.VMEM((_K, 1), jnp.float32)],
        interpret=interpret,
    )(z3d, codebook)
    return out.reshape(n_blocks * _M_BLOCK)


_DPAD = 128


@functools.cache
def _make_gather_kernel(npts):
    rows_per_w = npts // _NW // _GATHER_CHUNK
    mesh = plsc.VectorSubcoreMesh(core_axis_name="c", subcore_axis_name="s")

    @functools.partial(
        pl.kernel,
        mesh=mesh,
        out_type=jax.ShapeDtypeStruct(
            (npts // _GATHER_CHUNK, _GATHER_CHUNK, _DPAD), jnp.float32),
        scratch_types=[
            pltpu.VMEM((rows_per_w, _GATHER_CHUNK), jnp.int32),
            pltpu.VMEM((rows_per_w, _GATHER_CHUNK, _DPAD), jnp.float32),
            pltpu.SemaphoreType.DMA,
        ],
    )
    def _gather_kernel(idx_hbm, table_hbm, out_hbm, idx_v, rows_v, sem):
        wid = lax.axis_index("s") * _NC + lax.axis_index("c")
        base = wid * rows_per_w
        pltpu.sync_copy(idx_hbm.at[pl.ds(base, rows_per_w)], idx_v)
        for c in range(rows_per_w):
            pltpu.async_copy(table_hbm.at[idx_v.at[c]], rows_v.at[c],
                             sem).wait()
        pltpu.sync_copy(rows_v, out_hbm.at[pl.ds(base, rows_per_w)])

    return _gather_kernel


def _quantize_half(z3d_half, codebook, cb_pad):
    npts = z3d_half.shape[0] * _M_BLOCK
    idx = _encode_indices(z3d_half, codebook)
    idx2d = idx.reshape(npts // _GATHER_CHUNK, _GATHER_CHUNK)
    q = _make_gather_kernel(npts)(idx2d, cb_pad)
    q = q.reshape(npts, _DPAD)[:, :_D]
    nb = z3d_half.shape[0]
    return q.reshape(nb, _M_BLOCK, _D).transpose(0, 2, 1)


def kernel(z_e, codebook):
    b, c, h, w = z_e.shape
    z3d = z_e.reshape(b, c, h * w)
    cb_pad = jnp.pad(codebook, ((0, 0), (0, _DPAD - _D)))
    nh = b // 2
    qa = _quantize_half(z3d[:nh], codebook, cb_pad)
    qb = _quantize_half(z3d[nh:], codebook, cb_pad)
    return jnp.concatenate([qa, qb], axis=0).reshape(b, c, h, w)


# speedup vs baseline: 1.0636x; 1.0636x over previous
import functools

import jax
import jax.numpy as jnp
from jax import lax
from jax.experimental import pallas as pl
from jax.experimental.pallas import tpu as pltpu
from jax.experimental.pallas import tpu_sc as plsc

_NPTS = 16384
_D = 32
_K = 8192
_M_BLOCK = 1024
_DAUG = 40

_NC = 2
_NS = 16
_NW = _NC * _NS
_PTS_PER_W = _NPTS // _NW
_GATHER_CHUNK = 128
_ROWS_PER_W = _PTS_PER_W // _GATHER_CHUNK


def _argmin_body(z_ref, cb_ref, out_ref, cnorm_ref):
    @pl.when(pl.program_id(0) == 0)
    def _():
        cb = cb_ref[...]
        cnorm_ref[...] = jnp.sum(cb * cb, axis=1, keepdims=True) * 0.5

    s = lax.dot_general(cb_ref[...], z_ref[0], (((1,), (0,)), ((), ())),
                        preferred_element_type=jnp.float32)
    d = cnorm_ref[...] - s
    bi = jnp.argmin(d, axis=0)
    out_ref[...] = bi.astype(jnp.int32).reshape(1, 1, _M_BLOCK)


def _encode_indices(z3d, codebook, interpret=False):
    n_blocks = _NPTS // _M_BLOCK
    out = pl.pallas_call(
        _argmin_body,
        grid=(n_blocks,),
        in_specs=[
            pl.BlockSpec((1, _D, _M_BLOCK), lambda g: (g, 0, 0)),
            pl.BlockSpec((_K, _D), lambda g: (0, 0)),
        ],
        out_specs=pl.BlockSpec((1, 1, _M_BLOCK), lambda g: (g, 0, 0)),
        out_shape=jax.ShapeDtypeStruct((n_blocks, 1, _M_BLOCK), jnp.int32),
        scratch_shapes=[pltpu.VMEM((_K, 1), jnp.float32)],
        interpret=interpret,
    )(z3d, codebook)
    return out.reshape(_NPTS)


_DPAD = 128


@functools.cache
def _make_gather_kernel():
    mesh = plsc.VectorSubcoreMesh(core_axis_name="c", subcore_axis_name="s")

    @functools.partial(
        pl.kernel,
        mesh=mesh,
        out_type=jax.ShapeDtypeStruct(
            (_NPTS // _GATHER_CHUNK, _GATHER_CHUNK, _DPAD), jnp.float32),
        scratch_types=[
            pltpu.VMEM((_ROWS_PER_W, _GATHER_CHUNK), jnp.int32),
            pltpu.VMEM((_ROWS_PER_W, _GATHER_CHUNK, _DPAD), jnp.float32),
            pltpu.SemaphoreType.DMA,
        ],
    )
    def _gather_kernel(idx_hbm, table_hbm, out_hbm, idx_v, rows_v, sem):
        wid = lax.axis_index("s") * _NC + lax.axis_index("c")
        base = wid * _ROWS_PER_W
        pltpu.sync_copy(idx_hbm.at[pl.ds(base, _ROWS_PER_W)], idx_v)
        copies = [
            pltpu.async_copy(table_hbm.at[idx_v.at[c]], rows_v.at[c], sem)
            for c in range(_ROWS_PER_W)
        ]
        for cp in copies:
            cp.wait()
        pltpu.sync_copy(rows_v, out_hbm.at[pl.ds(base, _ROWS_PER_W)])

    return _gather_kernel


def kernel(z_e, codebook):
    b, c, h, w = z_e.shape
    z3d = z_e.reshape(b, c, h * w)
    idx = _encode_indices(z3d, codebook)
    idx2d = idx.reshape(_NPTS // _GATHER_CHUNK, _GATHER_CHUNK)
    cb_pad = jnp.pad(codebook, ((0, 0), (0, _DPAD - _D)))
    quantized = _make_gather_kernel()(idx2d, cb_pad)
    quantized = quantized.reshape(_NPTS, _DPAD)[:, :_D]
    q = quantized.reshape(b, h * w, c).transpose(0, 2, 1).reshape(b, c, h, w)
    return q


# speedup vs baseline: 1.0664x; 1.0026x over previous
import functools

import jax
import jax.numpy as jnp
from jax import lax
from jax.experimental import pallas as pl
from jax.experimental.pallas import tpu as pltpu
from jax.experimental.pallas import tpu_sc as plsc

_NPTS = 16384
_D = 32
_K = 8192
_M_BLOCK = 1024

_NC = 2
_NS = 16
_NW = _NC * _NS
_PTS_PER_W = _NPTS // _NW
_GATHER_CHUNK = 128
_ROWS_PER_W = _PTS_PER_W // _GATHER_CHUNK


def _argmin_body(z_ref, cb_ref, out_ref, cnorm_ref):
    @pl.when(pl.program_id(0) == 0)
    def _():
        cb = cb_ref[...]
        cnorm_ref[...] = jnp.sum(cb * cb, axis=1, keepdims=True)

    zb2 = z_ref[0] * -2.0
    s = lax.dot_general(cb_ref[...], zb2, (((1,), (0,)), ((), ())),
                        preferred_element_type=jnp.float32)
    d = cnorm_ref[...] + s
    bi = jnp.argmin(d, axis=0)
    out_ref[...] = bi.astype(jnp.int32).reshape(1, 1, _M_BLOCK)


def _encode_indices(z3d, codebook, interpret=False):
    n_blocks = _NPTS // _M_BLOCK
    out = pl.pallas_call(
        _argmin_body,
        grid=(n_blocks,),
        in_specs=[
            pl.BlockSpec((1, _D, _M_BLOCK), lambda g: (g, 0, 0)),
            pl.BlockSpec((_K, _D), lambda g: (0, 0)),
        ],
        out_specs=pl.BlockSpec((1, 1, _M_BLOCK), lambda g: (g, 0, 0)),
        out_shape=jax.ShapeDtypeStruct((n_blocks, 1, _M_BLOCK), jnp.int32),
        scratch_shapes=[pltpu.VMEM((_K, 1), jnp.float32)],
        interpret=interpret,
    )(z3d, codebook)
    return out.reshape(_NPTS)


_DPAD = 128


@functools.cache
def _make_gather_kernel():
    mesh = plsc.VectorSubcoreMesh(core_axis_name="c", subcore_axis_name="s")

    @functools.partial(
        pl.kernel,
        mesh=mesh,
        out_type=jax.ShapeDtypeStruct(
            (_NPTS // _GATHER_CHUNK, _GATHER_CHUNK, _DPAD), jnp.float32),
        scratch_types=[
            pltpu.VMEM((_ROWS_PER_W, _GATHER_CHUNK), jnp.int32),
            pltpu.VMEM((_ROWS_PER_W, _GATHER_CHUNK, _DPAD), jnp.float32),
            pltpu.SemaphoreType.DMA,
        ],
    )
    def _gather_kernel(idx_hbm, table_hbm, out_hbm, idx_v, rows_v, sem):
        wid = lax.axis_index("s") * _NC + lax.axis_index("c")
        base = wid * _ROWS_PER_W
        pltpu.sync_copy(idx_hbm.at[pl.ds(base, _ROWS_PER_W)], idx_v)
        for c in range(_ROWS_PER_W):
            pltpu.async_copy(table_hbm.at[idx_v.at[c]], rows_v.at[c],
                             sem).wait()
        pltpu.sync_copy(rows_v, out_hbm.at[pl.ds(base, _ROWS_PER_W)])

    return _gather_kernel


def kernel(z_e, codebook):
    b, c, h, w = z_e.shape
    z3d = z_e.reshape(b, c, h * w)
    idx = _encode_indices(z3d, codebook)
    idx2d = idx.reshape(_NPTS // _GATHER_CHUNK, _GATHER_CHUNK)
    cb_pad = jnp.pad(codebook, ((0, 0), (0, _DPAD - _D)))
    quantized = _make_gather_kernel()(idx2d, cb_pad)
    quantized = quantized.reshape(_NPTS, _DPAD)[:, :_D]
    q = quantized.reshape(b, h * w, c).transpose(0, 2, 1).reshape(b, c, h, w)
    return q
